# baseline (device time: 85131 ns/iter reference)
import os

import jax
import jax.numpy as jnp
from jax import lax
from jax.experimental import pallas as pl
from jax.experimental.pallas import tpu as pltpu

P = 16

_DISABLE_SEND = bool(int(os.environ.get("KERNEL_DISABLE_SEND", "0")))
_DISABLE_COMPUTE = bool(int(os.environ.get("KERNEL_DISABLE_COMPUTE", "0")))
_SKIP_WSTREAM = bool(int(os.environ.get("KERNEL_SKIP_WSTREAM", "0")))
_SKIP_DOT = bool(int(os.environ.get("KERNEL_SKIP_DOT", "0")))


def kernel(x, w_mat, scale_x, scale_w):
    M, K = x.shape
    N = w_mat.shape[1]
    NB = N // P
    SLOTS = 3

    def body(x_ref, w_ref, sx_ref, sw_ref, out_ref,
             xb_ref, wbuf, sendbuf, recvbuf, copy_sems, send_sems,
             recv_sems):
        my = lax.axis_index("i")
        s = sx_ref[0] * sw_ref[0]

        xb_ref[...] = x_ref[...].astype(jnp.bfloat16)

        def w_copy(d, slot):
            t = lax.rem(my + d, P)
            return pltpu.make_async_copy(
                w_ref.at[:, pl.ds(t * NB, NB)], wbuf.at[slot],
                copy_sems.at[slot],
            )

        def a2a_rdma(d):
            t = lax.rem(my + d, P)
            return pltpu.make_async_remote_copy(
                src_ref=sendbuf.at[d - 1],
                dst_ref=recvbuf.at[d - 1],
                send_sem=send_sems.at[d - 1],
                recv_sem=recv_sems.at[d - 1],
                device_id=(t,),
                device_id_type=pl.DeviceIdType.MESH,
            )

        def drain_recv(dd):
            src = lax.rem(my - dd + P, P)
            a2a_rdma(dd).wait_recv()
            out_ref[pl.ds(src * M, M), :] = recvbuf[dd - 1].astype(jnp.float32)

        LAG = 3

        if not _DISABLE_COMPUTE:
            if not _SKIP_WSTREAM:
                w_copy(0, 0).start()
                w_copy(1, 1).start()
            for d in range(P):
                if not _SKIP_WSTREAM:
                    if d + 2 < P:
                        w_copy(d + 2, (d + 2) % SLOTS).start()
                    w_copy(d, d % SLOTS).wait()
                if _SKIP_DOT:
                    blk = wbuf[d % SLOTS, :M, :].astype(jnp.float32)
                else:
                    wb = wbuf[d % SLOTS].astype(jnp.bfloat16)
                    blk = jnp.dot(
                        xb_ref[...], wb, preferred_element_type=jnp.float32
                    )
                blk = jnp.maximum(blk * s, 0.0)
                if d == 0:
                    out_ref[pl.ds(my * M, M), :] = blk
                else:
                    sendbuf[d - 1] = blk.astype(jnp.bfloat16)
                    if not _DISABLE_SEND:
                        a2a_rdma(d).start()
                if not _DISABLE_SEND and d > LAG:
                    drain_recv(d - LAG)
        else:
            out_ref[pl.ds(my * M, M), :] = jnp.zeros((M, NB), jnp.float32)
            for d in range(1, P):
                if not _DISABLE_SEND:
                    a2a_rdma(d).start()
            for d in range(1, P - LAG):
                drain_recv(d)

        if not _DISABLE_SEND:
            for dd in range(P - LAG, P):
                drain_recv(dd)
            for d in range(1, P):
                a2a_rdma(d).wait_send()

    return pl.pallas_call(
        body,
        out_shape=jax.ShapeDtypeStruct((P * M, NB), jnp.float32),
        in_specs=[
            pl.BlockSpec(memory_space=pltpu.VMEM),
            pl.BlockSpec(memory_space=pl.ANY),
            pl.BlockSpec(memory_space=pltpu.SMEM),
            pl.BlockSpec(memory_space=pltpu.SMEM),
        ],
        out_specs=pl.BlockSpec(memory_space=pltpu.VMEM),
        scratch_shapes=[
            pltpu.VMEM((M, K), jnp.bfloat16),
            pltpu.VMEM((SLOTS, K, NB), w_mat.dtype),
            pltpu.VMEM((P - 1, M, NB), jnp.bfloat16),
            pltpu.VMEM((P - 1, M, NB), jnp.bfloat16),
            pltpu.SemaphoreType.DMA((SLOTS,)),
            pltpu.SemaphoreType.DMA((P - 1,)),
            pltpu.SemaphoreType.DMA((P - 1,)),
        ],
        compiler_params=pltpu.CompilerParams(
            vmem_limit_bytes=56 * 1024 * 1024,
        ),
    )(x, w_mat, scale_x, scale_w)


# device time: 74219 ns/iter; 1.1470x vs baseline; 1.1470x over previous
import os

import jax
import jax.numpy as jnp
from jax import lax
from jax.experimental import pallas as pl
from jax.experimental.pallas import tpu as pltpu

P = 16

ORDER = [8, 1, 9, 2, 10, 3, 11, 4, 12, 5, 13, 6, 14, 7, 15]

_DISABLE_SEND = bool(int(os.environ.get("KERNEL_DISABLE_SEND", "0")))
_DISABLE_COMPUTE = bool(int(os.environ.get("KERNEL_DISABLE_COMPUTE", "0")))
_SKIP_WSTREAM = bool(int(os.environ.get("KERNEL_SKIP_WSTREAM", "0")))
_SKIP_DOT = bool(int(os.environ.get("KERNEL_SKIP_DOT", "0")))


def kernel(x, w_mat, scale_x, scale_w):
    M, K = x.shape
    N = w_mat.shape[1]
    NB = N // P
    SLOTS = 3

    def body(x_ref, w_ref, sx_ref, sw_ref, out_ref,
             xb_ref, wbuf, sendbuf, copy_sems, send_sems, recv_sems):
        my = lax.axis_index("i")
        s = sx_ref[0] * sw_ref[0]

        xb_ref[...] = x_ref[...].astype(jnp.bfloat16)

        def w_copy(d, slot):
            t = lax.rem(my + d, P)
            return pltpu.make_async_copy(
                w_ref.at[:, pl.ds(t * NB, NB)], wbuf.at[slot],
                copy_sems.at[slot],
            )

        def a2a_rdma(d):
            t = lax.rem(my + d, P)
            return pltpu.make_async_remote_copy(
                src_ref=sendbuf.at[d - 1],
                dst_ref=out_ref.at[pl.ds(my * M, M), :],
                send_sem=send_sems.at[d - 1],
                recv_sem=recv_sems.at[d - 1],
                device_id=(t,),
                device_id_type=pl.DeviceIdType.MESH,
            )

        steps = [0] + ORDER

        if not _DISABLE_COMPUTE:
            if not _SKIP_WSTREAM:
                w_copy(steps[0], 0).start()
                w_copy(steps[1], 1).start()
            for k, d in enumerate(steps):
                if not _SKIP_WSTREAM:
                    if k + 2 < P:
                        w_copy(steps[k + 2], (k + 2) % SLOTS).start()
                    w_copy(d, k % SLOTS).wait()
                if _SKIP_DOT:
                    blk = wbuf[k % SLOTS, :M, :] * s
                else:
                    wb = wbuf[k % SLOTS].astype(jnp.bfloat16)
                    blk = jnp.dot(
                        xb_ref[...], wb, preferred_element_type=jnp.float32
                    )
                    blk = blk * s
                blk = jnp.maximum(blk, 0.0).astype(jnp.bfloat16)
                if d == 0:
                    out_ref[pl.ds(my * M, M), :] = blk
                else:
                    sendbuf[d - 1] = blk
                    if not _DISABLE_SEND:
                        a2a_rdma(d).start()
        else:
            out_ref[pl.ds(my * M, M), :] = jnp.zeros((M, NB), jnp.bfloat16)
            for d in range(1, P):
                if not _DISABLE_SEND:
                    a2a_rdma(d).start()

        if not _DISABLE_SEND:
            for d in range(1, P):
                a2a_rdma(d).wait_recv()
            for d in range(1, P):
                a2a_rdma(d).wait_send()

    return pl.pallas_call(
        body,
        out_shape=jax.ShapeDtypeStruct((P * M, NB), jnp.bfloat16),
        in_specs=[
            pl.BlockSpec(memory_space=pltpu.VMEM),
            pl.BlockSpec(memory_space=pl.ANY),
            pl.BlockSpec(memory_space=pltpu.SMEM),
            pl.BlockSpec(memory_space=pltpu.SMEM),
        ],
        out_specs=pl.BlockSpec(memory_space=pltpu.VMEM),
        scratch_shapes=[
            pltpu.VMEM((M, K), jnp.bfloat16),
            pltpu.VMEM((SLOTS, K, NB), w_mat.dtype),
            pltpu.VMEM((P - 1, M, NB), jnp.bfloat16),
            pltpu.SemaphoreType.DMA((SLOTS,)),
            pltpu.SemaphoreType.DMA((P - 1,)),
            pltpu.SemaphoreType.DMA((P - 1,)),
        ],
        compiler_params=pltpu.CompilerParams(
            vmem_limit_bytes=56 * 1024 * 1024,
        ),
    )(x, w_mat, scale_x, scale_w)


# device time: 73500 ns/iter; 1.1582x vs baseline; 1.0098x over previous
import os

import jax
import jax.numpy as jnp
from jax import lax
from jax.experimental import pallas as pl
from jax.experimental.pallas import tpu as pltpu

P = 16

ORDER = [15, 8, 14, 7, 13, 6, 12, 5, 11, 4, 10, 3, 9, 2, 1]

_DISABLE_SEND = bool(int(os.environ.get("KERNEL_DISABLE_SEND", "0")))
_DISABLE_COMPUTE = bool(int(os.environ.get("KERNEL_DISABLE_COMPUTE", "0")))
_SKIP_WSTREAM = bool(int(os.environ.get("KERNEL_SKIP_WSTREAM", "0")))
_SKIP_DOT = bool(int(os.environ.get("KERNEL_SKIP_DOT", "0")))
_WIDE_COPY = int(os.environ.get("KERNEL_WIDE_COPY", "1"))


def kernel(x, w_mat, scale_x, scale_w):
    M, K = x.shape
    N = w_mat.shape[1]
    NB = N // P
    SLOTS = 4
    WB_COLS = _WIDE_COPY * NB if (_SKIP_DOT and _WIDE_COPY > 1) else NB

    def body(x_ref, w_ref, sx_ref, sw_ref, out_ref,
             xb_ref, wbuf, sendbuf, copy_sems, send_sems, recv_sems):
        my = lax.axis_index("i")
        s = sx_ref[0] * sw_ref[0]

        xb_ref[...] = x_ref[...].astype(jnp.bfloat16)

        def w_copy(d, slot):
            t = lax.rem(my + d, P)
            return pltpu.make_async_copy(
                w_ref.at[:, pl.ds(t * NB, NB)], wbuf.at[slot],
                copy_sems.at[slot],
            )

        def a2a_rdma(d):
            t = lax.rem(my + d, P)
            return pltpu.make_async_remote_copy(
                src_ref=sendbuf.at[d - 1],
                dst_ref=out_ref.at[pl.ds(my * M, M), :],
                send_sem=send_sems.at[d - 1],
                recv_sem=recv_sems.at[d - 1],
                device_id=(t,),
                device_id_type=pl.DeviceIdType.MESH,
            )

        steps = ORDER + [0]

        if _SKIP_DOT and _WIDE_COPY > 1:
            W = _WIDE_COPY * NB
            nq = N // W

            def wide_copy(q, slot):
                return pltpu.make_async_copy(
                    w_ref.at[:, pl.ds(q * W, W)], wbuf.at[slot],
                    copy_sems.at[slot],
                )

            wide_copy(0, 0).start()
            wide_copy(1, 1).start()
            for q in range(nq):
                if q + 2 < nq:
                    wide_copy(q + 2, (q + 2) % SLOTS).start()
                wide_copy(q, q % SLOTS).wait()
                out_ref[pl.ds(0, M), :] = (
                    wbuf[q % SLOTS, :M, :NB] * s).astype(jnp.bfloat16)
            return

        if not _DISABLE_COMPUTE:
            if not _SKIP_WSTREAM:
                w_copy(steps[0], 0).start()
                w_copy(steps[1], 1).start()
            for k, d in enumerate(steps):
                if not _SKIP_WSTREAM:
                    if k + 2 < P:
                        w_copy(steps[k + 2], (k + 2) % SLOTS).start()
                    w_copy(d, k % SLOTS).wait()
                if _SKIP_DOT:
                    blk = wbuf[k % SLOTS, :M, :] * s
                else:
                    wb = wbuf[k % SLOTS].astype(jnp.bfloat16)
                    blk = jnp.dot(
                        xb_ref[...], wb, preferred_element_type=jnp.float32
                    )
                    blk = blk * s
                blk = jnp.maximum(blk, 0.0).astype(jnp.bfloat16)
                if d == 0:
                    out_ref[pl.ds(my * M, M), :] = blk
                else:
                    sendbuf[d - 1] = blk
                    if not _DISABLE_SEND:
                        a2a_rdma(d).start()
        else:
            out_ref[pl.ds(my * M, M), :] = jnp.zeros((M, NB), jnp.bfloat16)
            for d in range(1, P):
                if not _DISABLE_SEND:
                    a2a_rdma(d).start()

        if not _DISABLE_SEND:
            for d in range(1, P):
                a2a_rdma(d).wait_recv()
            for d in range(1, P):
                a2a_rdma(d).wait_send()

    return pl.pallas_call(
        body,
        out_shape=jax.ShapeDtypeStruct((P * M, NB), jnp.bfloat16),
        in_specs=[
            pl.BlockSpec(memory_space=pltpu.VMEM),
            pl.BlockSpec(memory_space=pl.ANY),
            pl.BlockSpec(memory_space=pltpu.SMEM),
            pl.BlockSpec(memory_space=pltpu.SMEM),
        ],
        out_specs=pl.BlockSpec(memory_space=pltpu.VMEM),
        scratch_shapes=[
            pltpu.VMEM((M, K), jnp.bfloat16),
            pltpu.VMEM((SLOTS, K, WB_COLS), w_mat.dtype),
            pltpu.VMEM((P - 1, M, NB), jnp.bfloat16),
            pltpu.SemaphoreType.DMA((SLOTS,)),
            pltpu.SemaphoreType.DMA((P - 1,)),
            pltpu.SemaphoreType.DMA((P - 1,)),
        ],
        compiler_params=pltpu.CompilerParams(
            vmem_limit_bytes=56 * 1024 * 1024,
        ),
    )(x, w_mat, scale_x, scale_w)


# device time: 73391 ns/iter; 1.1600x vs baseline; 1.0015x over previous
import jax
import jax.numpy as jnp
from jax import lax
from jax.experimental import pallas as pl
from jax.experimental.pallas import tpu as pltpu

P = 16

ORDER = [15, 8, 14, 7, 13, 6, 12, 5, 11, 4, 10, 3, 9, 2, 1]


def kernel(x, w_mat, scale_x, scale_w):
    M, K = x.shape
    N = w_mat.shape[1]
    NB = N // P
    SLOTS = 4

    def body(x_ref, w_ref, sx_ref, sw_ref, out_ref,
             xb_ref, wbuf, sendbuf, copy_sems, send_sems, recv_sems):
        my = lax.axis_index("i")
        s = sx_ref[0] * sw_ref[0]

        xb_ref[...] = x_ref[...].astype(jnp.bfloat16)

        def w_copy(d, slot):
            t = lax.rem(my + d, P)
            return pltpu.make_async_copy(
                w_ref.at[:, pl.ds(t * NB, NB)], wbuf.at[slot],
                copy_sems.at[slot],
            )

        def a2a_rdma(d):
            t = lax.rem(my + d, P)
            return pltpu.make_async_remote_copy(
                src_ref=sendbuf.at[d - 1],
                dst_ref=out_ref.at[pl.ds(my * M, M), :],
                send_sem=send_sems.at[d - 1],
                recv_sem=recv_sems.at[d - 1],
                device_id=(t,),
                device_id_type=pl.DeviceIdType.MESH,
            )

        steps = ORDER + [0]

        w_copy(steps[0], 0).start()
        w_copy(steps[1], 1).start()
        for k, d in enumerate(steps):
            if k + 2 < P:
                w_copy(steps[k + 2], (k + 2) % SLOTS).start()
            w_copy(d, k % SLOTS).wait()
            wb = wbuf[k % SLOTS].astype(jnp.bfloat16)
            blk = jnp.dot(xb_ref[...], wb, preferred_element_type=jnp.float32)
            blk = jnp.maximum(blk * s, 0.0).astype(jnp.bfloat16)
            if d == 0:
                out_ref[pl.ds(my * M, M), :] = blk
            else:
                sendbuf[d - 1] = blk
                a2a_rdma(d).start()

        for d in range(1, P):
            a2a_rdma(d).wait_recv()
        for d in range(1, P):
            a2a_rdma(d).wait_send()

    return pl.pallas_call(
        body,
        out_shape=jax.ShapeDtypeStruct((P * M, NB), jnp.bfloat16),
        in_specs=[
            pl.BlockSpec(memory_space=pltpu.VMEM),
            pl.BlockSpec(memory_space=pl.ANY),
            pl.BlockSpec(memory_space=pltpu.SMEM),
            pl.BlockSpec(memory_space=pltpu.SMEM),
        ],
        out_specs=pl.BlockSpec(memory_space=pltpu.VMEM),
        scratch_shapes=[
            pltpu.VMEM((M, K), jnp.bfloat16),
            pltpu.VMEM((SLOTS, K, NB), w_mat.dtype),
            pltpu.VMEM((P - 1, M, NB), jnp.bfloat16),
            pltpu.SemaphoreType.DMA((SLOTS,)),
            pltpu.SemaphoreType.DMA((P - 1,)),
            pltpu.SemaphoreType.DMA((P - 1,)),
        ],
        compiler_params=pltpu.CompilerParams(
            vmem_limit_bytes=56 * 1024 * 1024,
        ),
    )(x, w_mat, scale_x, scale_w)
